# baseline (speedup 1.0000x reference)
import functools

import jax
import jax.numpy as jnp
from jax import lax
from jax.experimental import pallas as pl
from jax.experimental.pallas import tpu as pltpu
from jax.experimental.pallas import tpu_sc as plsc

MA_WINDOW = 100
CAPACITY = 0.5
C_KL = 1e-6

TC_CHUNK = 512


def _rowsum_xla(t):
    Tc, D = t.shape
    acc = t[:, 0:128]
    for j in range(1, D // 128):
        acc = acc + t[:, j * 128:(j + 1) * 128]
    c = acc[:, 0:8]
    for v in range(1, 16):
        c = c + acc[:, 8 * v:8 * v + 8]
    h = c[:, 0:4] + c[:, 4:8]
    h = h[:, 0:2] + h[:, 2:4]
    s = h[:, 0:1] + h[:, 1:2]
    return s[:, 0]


def _score_body(mu_p, mu_q, lv, mu_ch_p, mu_ch_q, lv_ch, d_st_ref, d_ch_ref):
    t_st = lv[...] + ((mu_p[...] - mu_q[...]) ** 2 + C_KL) * jnp.exp(-lv[...])
    t_ch = lv_ch[...] + ((mu_ch_p[...] - mu_ch_q[...]) ** 2 + C_KL) * jnp.exp(-lv_ch[...])
    d_st_ref[0, 0, :] = 0.5 * (_rowsum_xla(t_st) / t_st.shape[-1])
    d_ch_ref[0, 0, :] = 0.5 * (_rowsum_xla(t_ch) / t_ch.shape[-1])


def _scores(mu_p, mu_q, log_var_q, mu_ch_p, mu_ch_q, log_var_ch_q):
    B, T, D = mu_p.shape
    N = B * T
    n_steps = N // TC_CHUNK
    args = [x.reshape(N, D) for x in (mu_p, mu_q, log_var_q, mu_ch_p, mu_ch_q, log_var_ch_q)]
    in_spec = pl.BlockSpec((TC_CHUNK, D), lambda i: (i, 0))
    out_spec = pl.BlockSpec((1, 1, TC_CHUNK), lambda i: (i, 0, 0))
    d_st, d_ch = pl.pallas_call(
        _score_body,
        grid=(n_steps,),
        in_specs=[in_spec] * 6,
        out_specs=[out_spec, out_spec],
        out_shape=[jax.ShapeDtypeStruct((n_steps, 1, TC_CHUNK), jnp.float32)] * 2,
    )(*args)
    return d_st.reshape(B, T), d_ch.reshape(B, T)


def _sort_body(g_ref, dst_ref, d0_ref, mcu_ref, bcu_ref, vals_ref, idx_ref, scu_ref):
    R, C = 128, 128
    row = lax.broadcasted_iota(jnp.int32, (R, C), 0)
    lane = lax.broadcasted_iota(jnp.int32, (R, C), 1)
    row32 = row & 31
    flat = row32 * C + lane

    val = g_ref[...]
    idx = flat

    def partner(x, d):
        if d < C:
            lo = jnp.roll(x, -d, axis=1)
            hi = jnp.roll(x, d, axis=1)
            sel = (lane & d) == 0
        else:
            dr = d // C
            lo = jnp.roll(x, -dr, axis=0)
            hi = jnp.roll(x, dr, axis=0)
            sel = (row32 & dr) == 0
        return jnp.where(sel, lo, hi)

    for s in range(12):
        for sub in range(s, -1, -1):
            d = 1 << sub
            pv = partner(val, d)
            pix = partner(idx, d)
            self_first = (val > pv) | ((val == pv) & (idx < pix))
            is_upper = (flat & d) != 0
            asc = ((flat >> (s + 1)) & 1) == 1
            take_self = self_first != (is_upper != asc)
            val = jnp.where(take_self, val, pv)
            idx = jnp.where(take_self, idx, pix)

    vals_ref[...] = jnp.concatenate([val[b * 32:b * 32 + 16] for b in range(4)], axis=0)
    idx_ref[...] = jnp.concatenate([idx[b * 32:b * 32 + 16] for b in range(4)], axis=0)

    x = dst_ref[...]
    c = x
    for sh in (1, 2, 4, 8, 16, 32, 64):
        c = c + jnp.where(lane >= sh, jnp.roll(c, sh, axis=1), 0.0)
    rowtot = jnp.broadcast_to(c[:, C - 1:C], (R, C))
    acc = rowtot
    for sh in (1, 2, 4, 8, 16):
        acc = acc + jnp.where(row32 >= sh, jnp.roll(acc, sh, axis=0), 0.0)
    c_full = c + (acc - rowtot)

    W = MA_WINDOW
    shifted = jnp.where(lane >= W, jnp.roll(c_full, W, axis=1),
                        jnp.roll(jnp.roll(c_full, 1, axis=0), W - C, axis=1))
    d0 = d0_ref[...]
    ma = jnp.where(flat >= W,
                   (c_full - shifted) * (1.0 / W),
                   (c_full + (W - 1 - flat).astype(jnp.float32) * d0) * (1.0 / W))
    cu = x - mcu_ref[0, 0] * ma
    z = bcu_ref[0, 0] * cu
    s_cu = 1.0 / (1.0 + jnp.exp(-z))
    scu_ref[...] = jnp.sum(s_cu, axis=(0, 1), keepdims=True) * (1.0 / (R * C))


def _sort_topk(g, d_st, raw_m_cu, beta_cu):
    B, T = g.shape
    g128 = g.reshape(128, 128)
    d128 = d_st.reshape(128, 128)
    d0b = jnp.broadcast_to(d_st[:, :1], (B, T)).reshape(128, 128)
    mcu = jnp.asarray(raw_m_cu, jnp.float32).reshape(1, 1)
    bcu = jnp.asarray(beta_cu, jnp.float32).reshape(1, 1)
    vals, idx, scu = pl.pallas_call(
        _sort_body,
        out_shape=[jax.ShapeDtypeStruct((64, 128), jnp.float32),
                   jax.ShapeDtypeStruct((64, 128), jnp.int32),
                   jax.ShapeDtypeStruct((1, 1), jnp.float32)],
    )(g128, d128, d0b, mcu, bcu)
    return vals, idx, scu


def _make_gather(n_rows, D, rows_per_worker, chunk):
    mesh = plsc.VectorSubcoreMesh(core_axis_name="c", subcore_axis_name="s")
    info = plsc.get_sparse_core_info()
    NC = info.num_cores

    @functools.partial(
        pl.kernel, mesh=mesh,
        out_type=jax.ShapeDtypeStruct((n_rows, D), jnp.float32),
        scratch_types=[
            pltpu.VMEM((chunk,), jnp.int32),
            pltpu.VMEM((chunk, D), jnp.float32),
            pltpu.SemaphoreType.DMA,
        ],
    )
    def gather_k(table_hbm, idx_hbm, out_hbm, idx_v, rows_v, sem):
        wid = lax.axis_index("s") * NC + lax.axis_index("c")
        base = wid * rows_per_worker
        for j in range(rows_per_worker // chunk):
            off = base + j * chunk
            pltpu.sync_copy(idx_hbm.at[pl.ds(off, chunk)], idx_v)
            pltpu.async_copy(table_hbm.at[idx_v], rows_v, sem).wait()
            pltpu.sync_copy(rows_v, out_hbm.at[pl.ds(off, chunk)])

    return gather_k


def _gather(hidden_flat, gidx):
    n_rows = gidx.shape[0]
    D = hidden_flat.shape[1]
    k = _make_gather(n_rows, D, rows_per_worker=n_rows // 32, chunk=64)
    return k(hidden_flat, gidx)


def kernel(hidden_states, mu_p, mu_q, log_var_q, mu_ch_p, mu_ch_q, log_var_ch_q, beta_ce, beta_cu, raw_o_ce, raw_m_cu):
    B, T, D = hidden_states.shape
    k = min(max(1, int(T * CAPACITY)), T)

    D_st, D_ch = _scores(mu_p, mu_q, log_var_q, mu_ch_p, mu_ch_q, log_var_ch_q)
    CE = D_st - (D_ch - jnp.log(raw_o_ce + 1e-10))
    g_cont = jax.nn.sigmoid(beta_ce * CE)

    vals, idx_local, scu = _sort_topk(g_cont, D_st, raw_m_cu, beta_cu)

    topk_vals = vals.reshape(B, k)
    topk_idx = idx_local.reshape(B, k)
    gidx = (topk_idx + T * jnp.arange(B, dtype=jnp.int32)[:, None]).reshape(-1)

    selected_hidden = _gather(hidden_states.reshape(B * T, D), gidx)

    batch_idx = (jnp.arange(B * k, dtype=jnp.int32) // k)
    return (selected_hidden,
            batch_idx,
            topk_idx.reshape(-1),
            topk_vals.reshape(-1),
            scu[0, 0])


# speedup vs baseline: 1.1380x; 1.1380x over previous
import functools

import jax
import jax.numpy as jnp
from jax import lax
from jax.experimental import pallas as pl
from jax.experimental.pallas import tpu as pltpu
from jax.experimental.pallas import tpu_sc as plsc

MA_WINDOW = 100
CAPACITY = 0.5
C_KL = 1e-6

TC_CHUNK = 512


def _rowsum_xla(t):
    Tc, D = t.shape
    acc = t[:, 0:128]
    for j in range(1, D // 128):
        acc = acc + t[:, j * 128:(j + 1) * 128]
    tr = acc.T
    z = tr[0:8, :]
    for v in range(1, 16):
        z = z + tr[8 * v:8 * v + 8, :]
    r = z[0:4, :] + z[4:8, :]
    r = r[0:2, :] + r[2:4, :]
    r = r[0:1, :] + r[1:2, :]
    return r


def _score_body(mu_p, mu_q, lv, mu_ch_p, mu_ch_q, lv_ch, d_st_ref, d_ch_ref):
    t_st = lv[...] + ((mu_p[...] - mu_q[...]) ** 2 + C_KL) * jnp.exp(-lv[...])
    t_ch = lv_ch[...] + ((mu_ch_p[...] - mu_ch_q[...]) ** 2 + C_KL) * jnp.exp(-lv_ch[...])
    d_st_ref[0] = 0.5 * (_rowsum_xla(t_st) / t_st.shape[-1])
    d_ch_ref[0] = 0.5 * (_rowsum_xla(t_ch) / t_ch.shape[-1])


def _scores(mu_p, mu_q, log_var_q, mu_ch_p, mu_ch_q, log_var_ch_q):
    B, T, D = mu_p.shape
    N = B * T
    n_steps = N // TC_CHUNK
    args = [x.reshape(N, D) for x in (mu_p, mu_q, log_var_q, mu_ch_p, mu_ch_q, log_var_ch_q)]
    in_spec = pl.BlockSpec((TC_CHUNK, D), lambda i: (i, 0))
    out_spec = pl.BlockSpec((1, 1, TC_CHUNK), lambda i: (i, 0, 0))
    d_st, d_ch = pl.pallas_call(
        _score_body,
        grid=(n_steps,),
        in_specs=[in_spec] * 6,
        out_specs=[out_spec, out_spec],
        out_shape=[jax.ShapeDtypeStruct((n_steps, 1, TC_CHUNK), jnp.float32)] * 2,
    )(*args)
    return d_st.reshape(B, T), d_ch.reshape(B, T)


def _sort_body(g_ref, dst_ref, d0_ref, mcu_ref, bcu_ref, vals_ref, idx_ref, scu_ref):
    R, C = 128, 128
    row = lax.broadcasted_iota(jnp.int32, (R, C), 0)
    lane = lax.broadcasted_iota(jnp.int32, (R, C), 1)
    row32 = row & 31
    flat = row32 * C + lane

    val = g_ref[...]
    idx = flat

    def partner(x, d):
        if d < C:
            lo = jnp.roll(x, -d, axis=1)
            hi = jnp.roll(x, d, axis=1)
            sel = (lane & d) == 0
        else:
            dr = d // C
            lo = jnp.roll(x, -dr, axis=0)
            hi = jnp.roll(x, dr, axis=0)
            sel = (row32 & dr) == 0
        return jnp.where(sel, lo, hi)

    for s in range(12):
        for sub in range(s, -1, -1):
            d = 1 << sub
            pv = partner(val, d)
            pix = partner(idx, d)
            self_first = (val > pv) | ((val == pv) & (idx < pix))
            is_upper = (flat & d) != 0
            asc = ((flat >> (s + 1)) & 1) == 1
            take_self = self_first != (is_upper != asc)
            val = jnp.where(take_self, val, pv)
            idx = jnp.where(take_self, idx, pix)

    vals_ref[...] = jnp.concatenate([val[b * 32:b * 32 + 16] for b in range(4)], axis=0)
    idx_ref[...] = jnp.concatenate([idx[b * 32:b * 32 + 16] for b in range(4)], axis=0)

    x = dst_ref[...]
    c = x
    for sh in (1, 2, 4, 8, 16, 32, 64):
        c = c + jnp.where(lane >= sh, jnp.roll(c, sh, axis=1), 0.0)
    rowtot = jnp.broadcast_to(c[:, C - 1:C], (R, C))
    acc = rowtot
    for sh in (1, 2, 4, 8, 16):
        acc = acc + jnp.where(row32 >= sh, jnp.roll(acc, sh, axis=0), 0.0)
    c_full = c + (acc - rowtot)

    W = MA_WINDOW
    shifted = jnp.where(lane >= W, jnp.roll(c_full, W, axis=1),
                        jnp.roll(jnp.roll(c_full, 1, axis=0), W - C, axis=1))
    d0 = d0_ref[...]
    ma = jnp.where(flat >= W,
                   (c_full - shifted) * (1.0 / W),
                   (c_full + (W - 1 - flat).astype(jnp.float32) * d0) * (1.0 / W))
    cu = x - mcu_ref[0, 0] * ma
    z = bcu_ref[0, 0] * cu
    s_cu = 1.0 / (1.0 + jnp.exp(-z))
    scu_ref[...] = jnp.sum(s_cu, axis=(0, 1), keepdims=True) * (1.0 / (R * C))


def _sort_topk(g, d_st, raw_m_cu, beta_cu):
    B, T = g.shape
    g128 = g.reshape(128, 128)
    d128 = d_st.reshape(128, 128)
    d0b = jnp.broadcast_to(d_st[:, :1], (B, T)).reshape(128, 128)
    mcu = jnp.asarray(raw_m_cu, jnp.float32).reshape(1, 1)
    bcu = jnp.asarray(beta_cu, jnp.float32).reshape(1, 1)
    vals, idx, scu = pl.pallas_call(
        _sort_body,
        out_shape=[jax.ShapeDtypeStruct((64, 128), jnp.float32),
                   jax.ShapeDtypeStruct((64, 128), jnp.int32),
                   jax.ShapeDtypeStruct((1, 1), jnp.float32)],
    )(g128, d128, d0b, mcu, bcu)
    return vals, idx, scu


def _make_gather(n_rows, D, rows_per_worker, chunk):
    mesh = plsc.VectorSubcoreMesh(core_axis_name="c", subcore_axis_name="s")
    info = plsc.get_sparse_core_info()
    NC = info.num_cores

    @functools.partial(
        pl.kernel, mesh=mesh,
        out_type=jax.ShapeDtypeStruct((n_rows, D), jnp.float32),
        scratch_types=[
            pltpu.VMEM((chunk,), jnp.int32),
            pltpu.VMEM((chunk, D), jnp.float32),
            pltpu.SemaphoreType.DMA,
        ],
    )
    def gather_k(table_hbm, idx_hbm, out_hbm, idx_v, rows_v, sem):
        wid = lax.axis_index("s") * NC + lax.axis_index("c")
        base = wid * rows_per_worker
        for j in range(rows_per_worker // chunk):
            off = base + j * chunk
            pltpu.sync_copy(idx_hbm.at[pl.ds(off, chunk)], idx_v)
            pltpu.async_copy(table_hbm.at[idx_v], rows_v, sem).wait()
            pltpu.sync_copy(rows_v, out_hbm.at[pl.ds(off, chunk)])

    return gather_k


def _gather(hidden_flat, gidx):
    n_rows = gidx.shape[0]
    D = hidden_flat.shape[1]
    k = _make_gather(n_rows, D, rows_per_worker=n_rows // 32, chunk=64)
    return k(hidden_flat, gidx)


def kernel(hidden_states, mu_p, mu_q, log_var_q, mu_ch_p, mu_ch_q, log_var_ch_q, beta_ce, beta_cu, raw_o_ce, raw_m_cu):
    B, T, D = hidden_states.shape
    k = min(max(1, int(T * CAPACITY)), T)

    D_st, D_ch = _scores(mu_p, mu_q, log_var_q, mu_ch_p, mu_ch_q, log_var_ch_q)
    CE = D_st - (D_ch - jnp.log(raw_o_ce + 1e-10))
    g_cont = jax.nn.sigmoid(beta_ce * CE)

    vals, idx_local, scu = _sort_topk(g_cont, D_st, raw_m_cu, beta_cu)

    topk_vals = vals.reshape(B, k)
    topk_idx = idx_local.reshape(B, k)
    gidx = (topk_idx + T * jnp.arange(B, dtype=jnp.int32)[:, None]).reshape(-1)

    selected_hidden = _gather(hidden_states.reshape(B * T, D), gidx)

    batch_idx = (jnp.arange(B * k, dtype=jnp.int32) // k)
    return (selected_hidden,
            batch_idx,
            topk_idx.reshape(-1),
            topk_vals.reshape(-1),
            scu[0, 0])


# speedup vs baseline: 1.1489x; 1.0096x over previous
import functools

import jax
import jax.numpy as jnp
from jax import lax
from jax.experimental import pallas as pl
from jax.experimental.pallas import tpu as pltpu
from jax.experimental.pallas import tpu_sc as plsc

MA_WINDOW = 100
CAPACITY = 0.5
C_KL = 1e-6

TC_CHUNK = 512


def _rowsum_xla(t):
    Tc, D = t.shape
    acc = t[:, 0:128]
    for j in range(1, D // 128):
        acc = acc + t[:, j * 128:(j + 1) * 128]
    tr = acc.T
    z = tr[0:8, :]
    for v in range(1, 16):
        z = z + tr[8 * v:8 * v + 8, :]
    r = z[0:4, :] + z[4:8, :]
    r = r[0:2, :] + r[2:4, :]
    r = r[0:1, :] + r[1:2, :]
    return r


def _score_body(mu_p, mu_q, lv, mu_ch_p, mu_ch_q, lv_ch, d_st_ref, d_ch_ref):
    t_st = lv[...] + ((mu_p[...] - mu_q[...]) ** 2 + C_KL) * jnp.exp(-lv[...])
    t_ch = lv_ch[...] + ((mu_ch_p[...] - mu_ch_q[...]) ** 2 + C_KL) * jnp.exp(-lv_ch[...])
    d_st_ref[0] = 0.5 * (_rowsum_xla(t_st) / t_st.shape[-1])
    d_ch_ref[0] = 0.5 * (_rowsum_xla(t_ch) / t_ch.shape[-1])


def _scores(mu_p, mu_q, log_var_q, mu_ch_p, mu_ch_q, log_var_ch_q):
    B, T, D = mu_p.shape
    N = B * T
    n_steps = N // TC_CHUNK
    args = [x.reshape(N, D) for x in (mu_p, mu_q, log_var_q, mu_ch_p, mu_ch_q, log_var_ch_q)]
    in_spec = pl.BlockSpec((TC_CHUNK, D), lambda i: (i, 0))
    out_spec = pl.BlockSpec((1, 1, TC_CHUNK), lambda i: (i, 0, 0))
    d_st, d_ch = pl.pallas_call(
        _score_body,
        grid=(n_steps,),
        in_specs=[in_spec] * 6,
        out_specs=[out_spec, out_spec],
        out_shape=[jax.ShapeDtypeStruct((n_steps, 1, TC_CHUNK), jnp.float32)] * 2,
    )(*args)
    return d_st.reshape(B, T), d_ch.reshape(B, T)


def _sort_body(g_ref, dst_ref, d0_ref, mcu_ref, bcu_ref, vals_ref, idx_ref, scu_ref):
    R, C = 128, 128
    row = lax.broadcasted_iota(jnp.int32, (R, C), 0)
    lane = lax.broadcasted_iota(jnp.int32, (R, C), 1)
    row32 = row & 31
    flat = row32 * C + lane

    val = g_ref[...]
    idx = flat

    def partner(x, d):
        if d < C:
            lo = jnp.roll(x, -d, axis=1)
            hi = jnp.roll(x, d, axis=1)
            sel = (lane & d) == 0
        else:
            dr = d // C
            lo = jnp.roll(x, -dr, axis=0)
            hi = jnp.roll(x, dr, axis=0)
            sel = (row32 & dr) == 0
        return jnp.where(sel, lo, hi)

    for s in range(12):
        asc = ((flat >> (s + 1)) & 1) == 1
        for sub in range(s, -1, -1):
            d = 1 << sub
            pv = partner(val, d)
            pix = partner(idx, d)
            self_first = (val > pv) | ((val == pv) & (idx < pix))
            flip = ((flat & d) != 0) != asc
            take_self = self_first != flip
            val = jnp.where(take_self, val, pv)
            idx = jnp.where(take_self, idx, pix)

    vals_ref[...] = jnp.concatenate([val[b * 32:b * 32 + 16] for b in range(4)], axis=0)
    idx_ref[...] = jnp.concatenate([idx[b * 32:b * 32 + 16] for b in range(4)], axis=0)

    x = dst_ref[...]
    c = x
    for sh in (1, 2, 4, 8, 16, 32, 64):
        c = c + jnp.where(lane >= sh, jnp.roll(c, sh, axis=1), 0.0)
    rowtot = jnp.broadcast_to(c[:, C - 1:C], (R, C))
    acc = rowtot
    for sh in (1, 2, 4, 8, 16):
        acc = acc + jnp.where(row32 >= sh, jnp.roll(acc, sh, axis=0), 0.0)
    c_full = c + (acc - rowtot)

    W = MA_WINDOW
    shifted = jnp.where(lane >= W, jnp.roll(c_full, W, axis=1),
                        jnp.roll(jnp.roll(c_full, 1, axis=0), W - C, axis=1))
    d0 = d0_ref[...]
    ma = jnp.where(flat >= W,
                   (c_full - shifted) * (1.0 / W),
                   (c_full + (W - 1 - flat).astype(jnp.float32) * d0) * (1.0 / W))
    cu = x - mcu_ref[0, 0] * ma
    z = bcu_ref[0, 0] * cu
    s_cu = 1.0 / (1.0 + jnp.exp(-z))
    scu_ref[...] = jnp.sum(s_cu, axis=(0, 1), keepdims=True) * (1.0 / (R * C))


def _sort_topk(g, d_st, raw_m_cu, beta_cu):
    B, T = g.shape
    g128 = g.reshape(128, 128)
    d128 = d_st.reshape(128, 128)
    d0b = jnp.broadcast_to(d_st[:, :1], (B, T)).reshape(128, 128)
    mcu = jnp.asarray(raw_m_cu, jnp.float32).reshape(1, 1)
    bcu = jnp.asarray(beta_cu, jnp.float32).reshape(1, 1)
    vals, idx, scu = pl.pallas_call(
        _sort_body,
        out_shape=[jax.ShapeDtypeStruct((64, 128), jnp.float32),
                   jax.ShapeDtypeStruct((64, 128), jnp.int32),
                   jax.ShapeDtypeStruct((1, 1), jnp.float32)],
    )(g128, d128, d0b, mcu, bcu)
    return vals, idx, scu


def _make_gather(n_rows, D, rows_per_worker, chunk):
    mesh = plsc.VectorSubcoreMesh(core_axis_name="c", subcore_axis_name="s")
    info = plsc.get_sparse_core_info()
    NC = info.num_cores
    n_chunks = rows_per_worker // chunk

    @functools.partial(
        pl.kernel, mesh=mesh,
        out_type=jax.ShapeDtypeStruct((n_rows, D), jnp.float32),
        scratch_types=[
            pltpu.VMEM((rows_per_worker,), jnp.int32),
            pltpu.VMEM((chunk, D), jnp.float32),
            pltpu.VMEM((chunk, D), jnp.float32),
            pltpu.SemaphoreType.DMA,
            pltpu.SemaphoreType.DMA,
        ],
    )
    def gather_k(table_hbm, idx_hbm, out_hbm, idx_v, rows_a, rows_b, sem_a, sem_b):
        wid = lax.axis_index("s") * NC + lax.axis_index("c")
        base = wid * rows_per_worker
        pltpu.sync_copy(idx_hbm.at[pl.ds(base, rows_per_worker)], idx_v)
        bufs = (rows_a, rows_b)
        sems = (sem_a, sem_b)

        def gather_start(j):
            return pltpu.async_copy(
                table_hbm.at[idx_v.at[pl.ds(j * chunk, chunk)]],
                bufs[j % 2], sems[j % 2])

        cp = gather_start(0)
        for j in range(1, n_chunks):
            cp_next = gather_start(j)
            cp.wait()
            pltpu.sync_copy(bufs[(j - 1) % 2], out_hbm.at[pl.ds(base + (j - 1) * chunk, chunk)])
            cp = cp_next
        cp.wait()
        pltpu.sync_copy(bufs[(n_chunks - 1) % 2],
                        out_hbm.at[pl.ds(base + (n_chunks - 1) * chunk, chunk)])

    return gather_k


def _gather(hidden_flat, gidx):
    n_rows = gidx.shape[0]
    D = hidden_flat.shape[1]
    k = _make_gather(n_rows, D, rows_per_worker=n_rows // 32, chunk=32)
    return k(hidden_flat, gidx)


def kernel(hidden_states, mu_p, mu_q, log_var_q, mu_ch_p, mu_ch_q, log_var_ch_q, beta_ce, beta_cu, raw_o_ce, raw_m_cu):
    B, T, D = hidden_states.shape
    k = min(max(1, int(T * CAPACITY)), T)

    D_st, D_ch = _scores(mu_p, mu_q, log_var_q, mu_ch_p, mu_ch_q, log_var_ch_q)
    CE = D_st - (D_ch - jnp.log(raw_o_ce + 1e-10))
    g_cont = jax.nn.sigmoid(beta_ce * CE)

    vals, idx_local, scu = _sort_topk(g_cont, D_st, raw_m_cu, beta_cu)

    topk_vals = vals.reshape(B, k)
    topk_idx = idx_local.reshape(B, k)
    gidx = (topk_idx + T * jnp.arange(B, dtype=jnp.int32)[:, None]).reshape(-1)

    selected_hidden = _gather(hidden_states.reshape(B * T, D), gidx)

    batch_idx = (jnp.arange(B * k, dtype=jnp.int32) // k)
    return (selected_hidden,
            batch_idx,
            topk_idx.reshape(-1),
            topk_vals.reshape(-1),
            scu[0, 0])
